# baseline (speedup 1.0000x reference)
import jax
import jax.numpy as jnp
from jax.experimental import pallas as pl
from jax.experimental.pallas import tpu as pltpu


def _body(feat_ref, wm_ref, ws_ref, adj1_ref, adj2_ref, o1_ref, o2_ref,
          b1_ref, b2_ref):
    @pl.when(pl.program_id(0) == 0)
    def _prep():
        x = feat_ref[...].astype(jnp.bfloat16)
        xm = jnp.dot(x, wm_ref[...].astype(jnp.bfloat16),
                     preferred_element_type=jnp.float32)
        xs = jnp.dot(x, ws_ref[...].astype(jnp.bfloat16),
                     preferred_element_type=jnp.float32)
        miu = jnp.where(xm > 0, xm, jnp.exp(xm) - 1.0)
        sigma = jnp.maximum(xs, 0.0)
        att = jnp.exp(-sigma)
        b1_ref[...] = (miu * att).astype(jnp.bfloat16)
        b2_ref[...] = (sigma * att * att).astype(jnp.bfloat16)

    a1 = adj1_ref[...].astype(jnp.bfloat16)
    o1_ref[...] = jnp.dot(a1, b1_ref[...], preferred_element_type=jnp.float32)
    a2 = adj2_ref[...].astype(jnp.bfloat16)
    o2_ref[...] = jnp.dot(a2, b2_ref[...], preferred_element_type=jnp.float32)


@jax.jit
def kernel(features, adj_norm1, adj_norm2, W_miu, W_sigma):
    n, f = features.shape
    bm = 80 if n % 80 == 0 else 8
    out = pl.pallas_call(
        _body,
        grid=(n // bm,),
        in_specs=[
            pl.BlockSpec((n, f), lambda i: (0, 0)),
            pl.BlockSpec((f, f), lambda i: (0, 0)),
            pl.BlockSpec((f, f), lambda i: (0, 0)),
            pl.BlockSpec((bm, n), lambda i: (i, 0)),
            pl.BlockSpec((bm, n), lambda i: (i, 0)),
        ],
        out_specs=[
            pl.BlockSpec((bm, f), lambda i: (i, 0)),
            pl.BlockSpec((bm, f), lambda i: (i, 0)),
        ],
        out_shape=[
            jax.ShapeDtypeStruct((n, f), jnp.float32),
            jax.ShapeDtypeStruct((n, f), jnp.float32),
        ],
        scratch_shapes=[
            pltpu.VMEM((n, f), jnp.bfloat16),
            pltpu.VMEM((n, f), jnp.bfloat16),
        ],
    )(features, W_miu, W_sigma, adj_norm1, adj_norm2)
    return (out[0], out[1])


# speedup vs baseline: 1.0837x; 1.0837x over previous
import jax
import jax.numpy as jnp
from jax.experimental import pallas as pl
from jax.experimental.pallas import tpu as pltpu


def _body(feat_ref, wm_ref, ws_ref, adj1_ref, adj2_ref, o1_ref, o2_ref,
          b1_ref, b2_ref):
    @pl.when(pl.program_id(0) == 0)
    def _prep():
        x = feat_ref[...].astype(jnp.bfloat16)
        xm = jnp.dot(x, wm_ref[...].astype(jnp.bfloat16),
                     preferred_element_type=jnp.float32)
        xs = jnp.dot(x, ws_ref[...].astype(jnp.bfloat16),
                     preferred_element_type=jnp.float32)
        miu = jnp.where(xm > 0, xm, jnp.exp(xm) - 1.0)
        sigma = jnp.maximum(xs, 0.0)
        att = jnp.exp(-sigma)
        b1_ref[...] = (miu * att).astype(jnp.bfloat16)
        b2_ref[...] = (sigma * att * att).astype(jnp.bfloat16)

    a1 = adj1_ref[...].astype(jnp.bfloat16)
    o1_ref[...] = jnp.dot(a1, b1_ref[...], preferred_element_type=jnp.float32)
    a2 = adj2_ref[...].astype(jnp.bfloat16)
    o2_ref[...] = jnp.dot(a2, b2_ref[...], preferred_element_type=jnp.float32)


@jax.jit
def kernel(features, adj_norm1, adj_norm2, W_miu, W_sigma):
    n, f = features.shape
    bm = 304
    out = pl.pallas_call(
        _body,
        grid=((n + bm - 1) // bm,),
        in_specs=[
            pl.BlockSpec((n, f), lambda i: (0, 0)),
            pl.BlockSpec((f, f), lambda i: (0, 0)),
            pl.BlockSpec((f, f), lambda i: (0, 0)),
            pl.BlockSpec((bm, n), lambda i: (i, 0)),
            pl.BlockSpec((bm, n), lambda i: (i, 0)),
        ],
        out_specs=[
            pl.BlockSpec((bm, f), lambda i: (i, 0)),
            pl.BlockSpec((bm, f), lambda i: (i, 0)),
        ],
        out_shape=[
            jax.ShapeDtypeStruct((n, f), jnp.float32),
            jax.ShapeDtypeStruct((n, f), jnp.float32),
        ],
        scratch_shapes=[
            pltpu.VMEM((n, f), jnp.bfloat16),
            pltpu.VMEM((n, f), jnp.bfloat16),
        ],
        compiler_params=pltpu.CompilerParams(
            vmem_limit_bytes=112 * 1024 * 1024,
        ),
    )(features, W_miu, W_sigma, adj_norm1, adj_norm2)
    return (out[0], out[1])
